# baseline (speedup 1.0000x reference)
import dataclasses
import functools

import jax
import jax.numpy as jnp
from jax import lax
from jax.experimental import pallas as pl
from jax.experimental.pallas import tpu as pltpu
from jax.experimental.pallas import tpu_sc as plsc

B = 4096
F = 26
V = 1000
D = 16
NSLOT = F - 1
S = 512
WCOL = NSLOT * D
NC = 2
NS = 16
NW = NC * NS
BPW = B // NW
E = 4
IDXC = E * F
NG = 7
IDXP = NG * 16
NCHUNK = BPW // E


@functools.cache
def _get_sc_kernel():
    mesh = plsc.VectorSubcoreMesh(core_axis_name="c", subcore_axis_name="s")
    cp = pltpu.CompilerParams()
    if "needs_layout_passes" in pltpu.CompilerParams.__dataclass_fields__:
        cp = dataclasses.replace(cp, needs_layout_passes=False)
    if "use_tc_tiling_on_sc" in pltpu.CompilerParams.__dataclass_fields__:
        cp = dataclasses.replace(cp, use_tc_tiling_on_sc=True)
    return functools.partial(
        pl.kernel,
        out_type=jax.ShapeDtypeStruct((B,), jnp.float32),
        mesh=mesh,
        compiler_params=cp,
        scratch_types=[
            pltpu.VMEM((IDXP,), jnp.int32),
            pltpu.VMEM((IDXP,), jnp.int32),
            pltpu.VMEM((IDXC, S), jnp.float32),
            pltpu.VMEM((IDXC, S), jnp.float32),
            pltpu.VMEM((BPW,), jnp.float32),
            pltpu.SMEM((BPW,), jnp.float32),
            pltpu.SemaphoreType.DMA,
            pltpu.SemaphoreType.DMA,
        ],
    )(_ffm_sc)


def _ffm_sc(table, idx_hbm, out_hbm,
            idx0, idx1, rows0, rows1, outv, outs, sem0, sem1):
    wid = lax.axis_index("s") * NC + lax.axis_index("c")
    base_ex = wid * BPW

    lane16 = jax.lax.iota(jnp.int32, 16)
    wvec = [((lane16 + g * 16) % F) * V for g in range(NG)]

    zero16 = jnp.zeros((16,), jnp.int32)
    idx0[pl.ds(IDXC - 8, 16)] = zero16
    idx1[pl.ds(IDXC - 8, 16)] = zero16

    def start(chunk, idxb, rowb, sem):
        off = (base_ex + chunk * E) * F
        pltpu.sync_copy(idx_hbm.at[pl.ds(off, IDXC)], idxb.at[pl.ds(0, IDXC)])
        for g in range(NG):
            idxb[pl.ds(g * 16, 16)] = idxb[pl.ds(g * 16, 16)] + wvec[g]
        pltpu.async_copy(table.at[idxb.at[pl.ds(0, IDXC)]], rowb, sem)

    def wait(idxb, rowb, sem):
        pltpu.make_async_copy(table.at[idxb.at[pl.ds(0, IDXC)]], rowb,
                              sem).wait()

    def compute(chunk, rowb):
        @pl.loop(0, E)
        def _(e):
            r0 = e * F
            acc = jnp.zeros((D,), jnp.float32)
            for i in range(F):
                acc = acc + rowb[r0 + i, pl.ds(WCOL, D)]
            for i in range(F - 1):
                for j in range(i + 1, F):
                    a = rowb[r0 + i, pl.ds(D * (j - 1), D)]
                    b = rowb[r0 + j, pl.ds(D * i, D)]
                    acc = acc + a * b
            outs[chunk * E + e] = jnp.sum(acc, axis=0)

    start(0, idx0, rows0, sem0)

    @pl.loop(0, NCHUNK, step=2)
    def _(g):
        start(g + 1, idx1, rows1, sem1)
        wait(idx0, rows0, sem0)
        compute(g, rows0)

        @pl.when(g + 2 < NCHUNK)
        def _():
            start(g + 2, idx0, rows0, sem0)

        wait(idx1, rows1, sem1)
        compute(g + 1, rows1)

    @pl.loop(0, BPW // 16)
    def _(k):
        v = jnp.zeros((16,), jnp.float32)
        for l in range(16):
            v = jnp.where(lane16 == l, outs[k * 16 + l], v)
        outv[pl.ds(k * 16, 16)] = v

    pltpu.sync_copy(outv, out_hbm.at[pl.ds(base_ex, BPW)])


def kernel(indices, weights, embeddings, bias):
    t = jnp.transpose(embeddings, (0, 2, 1, 3)).reshape(F * V, NSLOT * D)
    table = jnp.concatenate(
        [t, weights.reshape(F * V, 1),
         jnp.zeros((F * V, S - WCOL - 1), jnp.float32)], axis=1)
    out = _get_sc_kernel()(table, indices.reshape(B * F))
    return out.reshape(B, 1) + bias
